# Initial kernel scaffold; baseline (speedup 1.0000x reference)
import jax
import jax.numpy as jnp
from jax.experimental import pallas as pl


def kernel(atom_features, bond_features, edge_index, bond_mol_ids, params):
    raise NotImplementedError("write your pallas kernel here")


# speedup vs baseline: 2.0564x; 2.0564x over previous
import functools

import jax
import jax.numpy as jnp
from jax import lax
from jax.experimental import pallas as pl
from jax.experimental.pallas import tpu as pltpu
from jax.experimental.pallas import tpu_sc as plsc

N = 10000
NPAD = 10240
E = 160000
H = 256
NUM_MOLS = 256
NC = 2
NS = 16
CH = 128
NCHUNKS = E // CH
HC = H // NC


def _tab_x_body(widths, x_ref, w_ref, b_ref, *outs):
    y = jnp.dot(x_ref[...], w_ref[...], preferred_element_type=jnp.float32)
    y = y + b_ref[0:1, :]
    o = 0
    for wd, r in zip(widths, outs):
        r[...] = y[:, o:o + wd]
        o += wd


def _tab_s_body(widths, ts_ref, agg_ref, w_ref, b_ref, *outs):
    ag = jnp.concatenate([agg_ref[0], agg_ref[1]], axis=1)
    x = jnp.maximum(ts_ref[...] + ag, 0.0)
    y = jnp.dot(x, w_ref[...], preferred_element_type=jnp.float32)
    y = y + b_ref[0:1, :]
    o = 0
    for wd, r in zip(widths, outs):
        r[...] = y[:, o:o + wd]
        o += wd


def _tables_call(first, x_or_ts, agg, w, b2, widths):
    BN = 1024
    grid = (NPAD // BN,)
    dout = sum(widths)
    k = x_or_ts.shape[1]
    out_shape = [jax.ShapeDtypeStruct((NPAD, wd), jnp.float32) for wd in widths]
    out_specs = [pl.BlockSpec((BN, wd), lambda i: (i, 0)) for wd in widths]
    if first:
        in_specs = [pl.BlockSpec((BN, k), lambda i: (i, 0))]
        args = (x_or_ts,)
        body = functools.partial(_tab_x_body, tuple(widths))
    else:
        in_specs = [pl.BlockSpec((BN, H), lambda i: (i, 0)),
                    pl.BlockSpec((NC, BN, HC), lambda i: (0, i, 0))]
        args = (x_or_ts, agg)
        body = functools.partial(_tab_s_body, tuple(widths))
    in_specs += [pl.BlockSpec((k, dout), lambda i: (0, 0)),
                 pl.BlockSpec((8, dout), lambda i: (0, 0))]
    return pl.pallas_call(body, grid=grid, in_specs=in_specs,
                          out_specs=out_specs, out_shape=out_shape)(*args, w, b2)


def _edges_body(hb_ref, w_ref, b_ref, gsn_ref, gw_ref, nb_ref, msg_ref):
    eh = jnp.dot(hb_ref[...], w_ref[...], preferred_element_type=jnp.float32)
    s = eh + b_ref[0:1, :] + gsn_ref[:, :H] + gw_ref[...]
    nb_ref[...] = jnp.maximum(s, 0.0)
    m = jax.nn.sigmoid(s) * gsn_ref[:, H:]
    msg_ref[0] = m[:, :HC]
    msg_ref[1] = m[:, HC:]


def _edges_call(hb, w, b2, gsn, gw):
    BE = 1000
    grid = (E // BE,)
    k = hb.shape[1]
    out_shape = [jax.ShapeDtypeStruct((E, H), jnp.float32),
                 jax.ShapeDtypeStruct((NC, E, HC), jnp.float32)]
    out_specs = [pl.BlockSpec((BE, H), lambda i: (i, 0)),
                 pl.BlockSpec((NC, BE, HC), lambda i: (0, i, 0))]
    in_specs = [pl.BlockSpec((BE, k), lambda i: (i, 0)),
                pl.BlockSpec((k, H), lambda i: (0, 0)),
                pl.BlockSpec((8, H), lambda i: (0, 0)),
                pl.BlockSpec((BE, 2 * H), lambda i: (i, 0)),
                pl.BlockSpec((BE, H), lambda i: (i, 0))]
    return pl.pallas_call(_edges_body, grid=grid, in_specs=in_specs,
                          out_specs=out_specs, out_shape=out_shape)(hb, w, b2, gsn, gw)


def _fedges_body(hb_ref, w_ref, b_ref, gv_ref, gw_ref, ids_ref, out_ref, cnt_ref):
    i = pl.program_id(0)
    y = jnp.dot(hb_ref[...], w_ref[...], preferred_element_type=jnp.float32)
    y = y + b_ref[0:1, :]
    s = y[:, :H] + gv_ref[...] + gw_ref[...]
    m = jax.nn.sigmoid(s) * y[:, H:]
    out_ref[0] = m[:, :HC]
    out_ref[1] = m[:, HC:]
    be = ids_ref.shape[0]
    oh = (ids_ref[...] == jax.lax.broadcasted_iota(jnp.int32, (be, NUM_MOLS), 1))
    cnt = jnp.dot(oh.astype(jnp.float32).T, jnp.ones((be, 8), jnp.float32),
                  preferred_element_type=jnp.float32)

    @pl.when(i == 0)
    def _():
        cnt_ref[...] = jnp.zeros_like(cnt_ref)

    cnt_ref[...] += cnt


def _fedges_call(hb, w, b2, gv, gw, ids):
    BE = 1000
    grid = (E // BE,)
    out_shape = [jax.ShapeDtypeStruct((NC, E, HC), jnp.float32),
                 jax.ShapeDtypeStruct((NUM_MOLS, 8), jnp.float32)]
    out_specs = [pl.BlockSpec((NC, BE, HC), lambda i: (0, i, 0)),
                 pl.BlockSpec((NUM_MOLS, 8), lambda i: (0, 0))]
    in_specs = [pl.BlockSpec((BE, H), lambda i: (i, 0)),
                pl.BlockSpec((H, 2 * H), lambda i: (0, 0)),
                pl.BlockSpec((8, 2 * H), lambda i: (0, 0)),
                pl.BlockSpec((BE, H), lambda i: (i, 0)),
                pl.BlockSpec((BE, H), lambda i: (i, 0)),
                pl.BlockSpec((BE, 1), lambda i: (i, 0))]
    return pl.pallas_call(_fedges_body, grid=grid, in_specs=in_specs,
                          out_specs=out_specs, out_shape=out_shape)(
                              hb, w, b2, gv, gw, ids)


def _div_body(sums_ref, cnt_ref, out_ref):
    c = jnp.maximum(cnt_ref[:, 0:1], 1.0)
    out_ref[:, :HC] = sums_ref[0] / c
    out_ref[:, HC:] = sums_ref[1] / c


def _div_call(sums3, counts):
    return pl.pallas_call(
        _div_body,
        out_shape=jax.ShapeDtypeStruct((NUM_MOLS, H), jnp.float32),
    )(sums3, counts)


def _sc_gather(t1, t2, idx1, idx2, d1, d2):
    mesh = plsc.VectorSubcoreMesh(core_axis_name="c", subcore_axis_name="s")

    @functools.partial(
        pl.kernel, mesh=mesh,
        out_type=[jax.ShapeDtypeStruct((E, d1), jnp.float32),
                  jax.ShapeDtypeStruct((E, d2), jnp.float32)],
        scratch_types=[pltpu.VMEM((CH,), jnp.int32),
                       pltpu.VMEM((CH,), jnp.int32),
                       pltpu.VMEM((CH, d1), jnp.float32),
                       pltpu.VMEM((CH, d2), jnp.float32),
                       pltpu.SemaphoreType.DMA],
    )
    def k(t1_hbm, t2_hbm, i1_hbm, i2_hbm, o1_hbm, o2_hbm,
          i1_v, i2_v, b1_v, b2_v, sem):
        cid = lax.axis_index("c")
        sid = lax.axis_index("s")
        wid = sid * NC + cid
        nw = NC * NS

        def body(i, carry):
            chunk = wid + i * nw

            @pl.when(chunk < NCHUNKS)
            def _():
                base = chunk * CH
                pltpu.sync_copy(i1_hbm.at[pl.ds(base, CH)], i1_v)
                pltpu.sync_copy(i2_hbm.at[pl.ds(base, CH)], i2_v)
                pltpu.async_copy(t1_hbm.at[i1_v], b1_v, sem).wait()
                pltpu.async_copy(t2_hbm.at[i2_v], b2_v, sem).wait()
                pltpu.sync_copy(b1_v, o1_hbm.at[pl.ds(base, CH)])
                pltpu.sync_copy(b2_v, o2_hbm.at[pl.ds(base, CH)])
            return carry

        lax.fori_loop(0, (NCHUNKS + nw - 1) // nw, body, 0)

    return k(t1, t2, idx1, idx2)


def _sc_scatter(msg3, dstv):
    mesh = plsc.VectorSubcoreMesh(core_axis_name="c", subcore_axis_name="s")
    rows_per_sub = NPAD // NS

    @functools.partial(
        pl.kernel, mesh=mesh,
        out_type=jax.ShapeDtypeStruct((NC, NPAD, HC), jnp.float32),
        scratch_types=[pltpu.VMEM((CH,), jnp.int32),
                       pltpu.VMEM((CH, HC), jnp.float32),
                       pltpu.VMEM_SHARED((NPAD, HC), jnp.float32),
                       pltpu.SemaphoreType.DMA],
    )
    def k(msg_hbm, dst_hbm, agg_hbm, idx_v, buf_v, acc_sh, sem):
        cid = lax.axis_index("c")
        sid = lax.axis_index("s")
        zer = jnp.zeros((16,), jnp.float32)

        def zrow(r, carry):
            for j in range(HC // 16):
                buf_v[r, j * 16:(j + 1) * 16] = zer
            return carry

        lax.fori_loop(0, CH, zrow, 0)

        def zcp(kk, carry):
            pltpu.sync_copy(buf_v, acc_sh.at[pl.ds(sid * rows_per_sub + kk * CH, CH)])
            return carry

        lax.fori_loop(0, rows_per_sub // CH, zcp, 0)
        plsc.subcore_barrier()

        def body(i, carry):
            chunk = sid + i * NS

            @pl.when(chunk < NCHUNKS)
            def _():
                base = chunk * CH
                pltpu.sync_copy(dst_hbm.at[pl.ds(base, CH)], idx_v)
                pltpu.sync_copy(msg_hbm.at[cid, pl.ds(base, CH)], buf_v)
                pltpu.sync_copy(buf_v, acc_sh.at[idx_v], add=True)
            return carry

        lax.fori_loop(0, (NCHUNKS + NS - 1) // NS, body, 0)
        plsc.subcore_barrier()

        def flsh(kk, carry):
            r0 = sid * rows_per_sub + kk * CH
            pltpu.sync_copy(acc_sh.at[pl.ds(r0, CH)], buf_v)
            pltpu.sync_copy(buf_v, agg_hbm.at[cid, pl.ds(r0, CH)])
            return carry

        lax.fori_loop(0, rows_per_sub // CH, flsh, 0)

    return k(msg3, dstv)


def _sc_scatter_mols(gated3, ids):
    mesh = plsc.VectorSubcoreMesh(core_axis_name="c", subcore_axis_name="s")

    @functools.partial(
        pl.kernel, mesh=mesh,
        out_type=jax.ShapeDtypeStruct((NC, NUM_MOLS, HC), jnp.float32),
        scratch_types=[pltpu.VMEM((CH,), jnp.int32),
                       pltpu.VMEM((CH, HC), jnp.float32),
                       pltpu.VMEM_SHARED((NUM_MOLS, HC), jnp.float32),
                       pltpu.SemaphoreType.DMA],
    )
    def k(g_hbm, ids_hbm, sums_hbm, idx_v, buf_v, acc_sh, sem):
        cid = lax.axis_index("c")
        sid = lax.axis_index("s")
        zer = jnp.zeros((16,), jnp.float32)

        def zrow(r, carry):
            for j in range(HC // 16):
                buf_v[r, j * 16:(j + 1) * 16] = zer
            return carry

        lax.fori_loop(0, CH, zrow, 0)

        @pl.when(sid < NUM_MOLS // CH)
        def _():
            pltpu.sync_copy(buf_v, acc_sh.at[pl.ds(sid * CH, CH)])

        plsc.subcore_barrier()

        def body(i, carry):
            chunk = sid + i * NS

            @pl.when(chunk < NCHUNKS)
            def _():
                base = chunk * CH
                pltpu.sync_copy(ids_hbm.at[pl.ds(base, CH)], idx_v)
                pltpu.sync_copy(g_hbm.at[cid, pl.ds(base, CH)], buf_v)
                pltpu.sync_copy(buf_v, acc_sh.at[idx_v], add=True)
            return carry

        lax.fori_loop(0, (NCHUNKS + NS - 1) // NS, body, 0)
        plsc.subcore_barrier()

        @pl.when(sid < NUM_MOLS // CH)
        def _():
            pltpu.sync_copy(acc_sh.at[pl.ds(sid * CH, CH)], buf_v)
            pltpu.sync_copy(buf_v, sums_hbm.at[cid, pl.ds(sid * CH, CH)])

    return k(gated3, ids)


def _b2(b):
    return jnp.tile(b[None, :], (8, 1))


def kernel(atom_features, bond_features, edge_index, bond_mol_ids, params):
    src = edge_index[0]
    dst = edge_index[1]
    x0 = jnp.pad(atom_features, ((0, NPAD - N), (0, 128 - atom_features.shape[1])))
    hb = jnp.pad(bond_features, ((0, 0), (0, 128 - bond_features.shape[1])))
    ts = None
    agg = None
    for li, lp in enumerate(params["layers"]):
        wcat = jnp.concatenate([lp["V"]["w"], lp["W_nei"]["w"],
                                lp["W"]["w"], lp["W_self"]["w"]], axis=1)
        bcat = jnp.concatenate([lp["V"]["b"], lp["W_nei"]["b"],
                                lp["W"]["b"], lp["W_self"]["b"]])
        if li == 0:
            wcat = jnp.pad(wcat, ((0, 128 - wcat.shape[0]), (0, 0)))
            tsn, tw, tself = _tables_call(True, x0, None, wcat, _b2(bcat),
                                          (2 * H, H, H))
        else:
            tsn, tw, tself = _tables_call(False, ts, agg, wcat, _b2(bcat),
                                          (2 * H, H, H))
        gsn, gw = _sc_gather(tsn, tw, src, dst, 2 * H, H)
        wb = lp["W_bond"]["w"]
        if li == 0:
            wb = jnp.pad(wb, ((0, 128 - wb.shape[0]), (0, 0)))
        nb, msg3 = _edges_call(hb, wb, _b2(lp["W_bond"]["b"]), gsn, gw)
        agg = _sc_scatter(msg3, dst)
        ts = tself
        hb = nb
    wvw = jnp.concatenate([params["V"]["w"], params["W"]["w"]], axis=1)
    bvw = jnp.concatenate([params["V"]["b"], params["W"]["b"]])
    tv, tw2 = _tables_call(False, ts, agg, wvw, _b2(bvw), (H, H))
    gv, gw2 = _sc_gather(tv, tw2, src, dst, H, H)
    wua = jnp.concatenate([params["U"]["w"], params["A"]["w"]], axis=1)
    bua = jnp.concatenate([params["U"]["b"], params["A"]["b"]])
    gated3, counts = _fedges_call(hb, wua, _b2(bua), gv, gw2,
                                  bond_mol_ids[:, None])
    sums3 = _sc_scatter_mols(gated3, bond_mol_ids)
    return _div_call(sums3, counts)


# speedup vs baseline: 2.5864x; 1.2577x over previous
import functools

import jax
import jax.numpy as jnp
from jax import lax
from jax.experimental import pallas as pl
from jax.experimental.pallas import tpu as pltpu
from jax.experimental.pallas import tpu_sc as plsc

N = 10000
NPAD = 10240
E = 160000
H = 256
NUM_MOLS = 256
NC = 2
NS = 16
CH = 128
NCHUNKS = E // CH
HC = H // NC


def _pack2(a, b):
    ua = jax.lax.bitcast_convert_type(a, jnp.uint32)
    ub = jax.lax.bitcast_convert_type(b, jnp.uint32)
    ra = (ua + jnp.uint32(0x7FFF) + ((ua >> 16) & jnp.uint32(1))) >> 16
    rb = (ub + jnp.uint32(0x7FFF) + ((ub >> 16) & jnp.uint32(1))) & jnp.uint32(0xFFFF0000)
    return jax.lax.bitcast_convert_type(ra | rb, jnp.int32)


def _unlo(x):
    u = jax.lax.bitcast_convert_type(x, jnp.uint32)
    return jax.lax.bitcast_convert_type(u << 16, jnp.float32)


def _unhi(x):
    u = jax.lax.bitcast_convert_type(x, jnp.uint32)
    return jax.lax.bitcast_convert_type(u & jnp.uint32(0xFFFF0000), jnp.float32)


def _tab_body(first, final, *refs):
    if first:
        x_ref, w_ref, b_ref = refs[:3]
        outs = refs[3:]
        x = x_ref[...]
    else:
        ts_ref, agg_ref, w_ref, b_ref = refs[:4]
        outs = refs[4:]
        ag = jnp.concatenate([agg_ref[0], agg_ref[1]], axis=1)
        x = jnp.maximum(ts_ref[...] + ag, 0.0)
    y = jnp.dot(x, w_ref[...], preferred_element_type=jnp.float32)
    y = y + b_ref[0:1, :]
    if final:
        v = y[:, :H]
        w = y[:, H:]
        outs[0][...] = _pack2(v[:, :HC], v[:, HC:])
        outs[1][...] = _pack2(w[:, :HC], w[:, HC:])
    else:
        outs[0][...] = _pack2(y[:, :H], y[:, H:2 * H])
        outs[1][...] = _pack2(y[:, 2 * H:2 * H + HC], y[:, 2 * H + HC:3 * H])
        outs[2][...] = y[:, 3 * H:]


def _tables_call(first, final, x_or_ts, agg, w, b2):
    BN = 1024
    grid = (NPAD // BN,)
    dout = w.shape[1]
    k = x_or_ts.shape[1]
    if final:
        out_shape = [jax.ShapeDtypeStruct((NPAD, HC), jnp.int32),
                     jax.ShapeDtypeStruct((NPAD, HC), jnp.int32)]
        out_specs = [pl.BlockSpec((BN, HC), lambda i: (i, 0)),
                     pl.BlockSpec((BN, HC), lambda i: (i, 0))]
    else:
        out_shape = [jax.ShapeDtypeStruct((NPAD, H), jnp.int32),
                     jax.ShapeDtypeStruct((NPAD, HC), jnp.int32),
                     jax.ShapeDtypeStruct((NPAD, H), jnp.float32)]
        out_specs = [pl.BlockSpec((BN, H), lambda i: (i, 0)),
                     pl.BlockSpec((BN, HC), lambda i: (i, 0)),
                     pl.BlockSpec((BN, H), lambda i: (i, 0))]
    if first:
        in_specs = [pl.BlockSpec((BN, k), lambda i: (i, 0))]
        args = (x_or_ts,)
    else:
        in_specs = [pl.BlockSpec((BN, H), lambda i: (i, 0)),
                    pl.BlockSpec((NC, BN, HC), lambda i: (0, i, 0))]
        args = (x_or_ts, agg)
    in_specs += [pl.BlockSpec((k, dout), lambda i: (0, 0)),
                 pl.BlockSpec((8, dout), lambda i: (0, 0))]
    body = functools.partial(_tab_body, first, final)
    return pl.pallas_call(body, grid=grid, in_specs=in_specs,
                          out_specs=out_specs, out_shape=out_shape)(*args, w, b2)


def _edges_body(hb_ref, w_ref, b_ref, gvn_ref, gw_ref, nb_ref, msg_ref):
    eh = jnp.dot(hb_ref[...], w_ref[...], preferred_element_type=jnp.float32)
    vn = gvn_ref[...]
    wx = jnp.concatenate([_unlo(gw_ref[...]), _unhi(gw_ref[...])], axis=1)
    s = eh + b_ref[0:1, :] + _unlo(vn) + wx
    nb_ref[...] = jnp.maximum(s, 0.0)
    m = jax.nn.sigmoid(s) * _unhi(vn)
    msg_ref[0] = m[:, :HC]
    msg_ref[1] = m[:, HC:]


def _edges_call(hb, w, b2, gvn, gw):
    BE = 1000
    grid = (E // BE,)
    k = hb.shape[1]
    out_shape = [jax.ShapeDtypeStruct((E, H), jnp.float32),
                 jax.ShapeDtypeStruct((NC, E, HC), jnp.float32)]
    out_specs = [pl.BlockSpec((BE, H), lambda i: (i, 0)),
                 pl.BlockSpec((NC, BE, HC), lambda i: (0, i, 0))]
    in_specs = [pl.BlockSpec((BE, k), lambda i: (i, 0)),
                pl.BlockSpec((k, H), lambda i: (0, 0)),
                pl.BlockSpec((8, H), lambda i: (0, 0)),
                pl.BlockSpec((BE, H), lambda i: (i, 0)),
                pl.BlockSpec((BE, HC), lambda i: (i, 0))]
    return pl.pallas_call(_edges_body, grid=grid, in_specs=in_specs,
                          out_specs=out_specs, out_shape=out_shape)(hb, w, b2, gvn, gw)


def _fedges_body(hb_ref, w_ref, b_ref, gv_ref, gw_ref, ids_ref, out_ref, cnt_ref):
    i = pl.program_id(0)
    y = jnp.dot(hb_ref[...], w_ref[...], preferred_element_type=jnp.float32)
    y = y + b_ref[0:1, :]
    gv = jnp.concatenate([_unlo(gv_ref[...]), _unhi(gv_ref[...])], axis=1)
    gw = jnp.concatenate([_unlo(gw_ref[...]), _unhi(gw_ref[...])], axis=1)
    s = y[:, :H] + gv + gw
    m = jax.nn.sigmoid(s) * y[:, H:]
    out_ref[0] = m[:, :HC]
    out_ref[1] = m[:, HC:]
    be = ids_ref.shape[0]
    oh = (ids_ref[...] == jax.lax.broadcasted_iota(jnp.int32, (be, NUM_MOLS), 1))
    cnt = jnp.dot(oh.astype(jnp.float32).T, jnp.ones((be, 8), jnp.float32),
                  preferred_element_type=jnp.float32)

    @pl.when(i == 0)
    def _():
        cnt_ref[...] = jnp.zeros_like(cnt_ref)

    cnt_ref[...] += cnt


def _fedges_call(hb, w, b2, gv, gw, ids):
    BE = 1000
    grid = (E // BE,)
    out_shape = [jax.ShapeDtypeStruct((NC, E, HC), jnp.float32),
                 jax.ShapeDtypeStruct((NUM_MOLS, 8), jnp.float32)]
    out_specs = [pl.BlockSpec((NC, BE, HC), lambda i: (0, i, 0)),
                 pl.BlockSpec((NUM_MOLS, 8), lambda i: (0, 0))]
    in_specs = [pl.BlockSpec((BE, H), lambda i: (i, 0)),
                pl.BlockSpec((H, 2 * H), lambda i: (0, 0)),
                pl.BlockSpec((8, 2 * H), lambda i: (0, 0)),
                pl.BlockSpec((BE, HC), lambda i: (i, 0)),
                pl.BlockSpec((BE, HC), lambda i: (i, 0)),
                pl.BlockSpec((BE, 1), lambda i: (i, 0))]
    return pl.pallas_call(_fedges_body, grid=grid, in_specs=in_specs,
                          out_specs=out_specs, out_shape=out_shape)(
                              hb, w, b2, gv, gw, ids)


def _div_body(sums_ref, cnt_ref, out_ref):
    c = jnp.maximum(cnt_ref[:, 0:1], 1.0)
    out_ref[:, :HC] = sums_ref[0] / c
    out_ref[:, HC:] = sums_ref[1] / c


def _div_call(sums3, counts):
    return pl.pallas_call(
        _div_body,
        out_shape=jax.ShapeDtypeStruct((NUM_MOLS, H), jnp.float32),
    )(sums3, counts)


def _sc_gather(t1, t2, idx1, idx2, d1, d2):
    mesh = plsc.VectorSubcoreMesh(core_axis_name="c", subcore_axis_name="s")

    @functools.partial(
        pl.kernel, mesh=mesh,
        out_type=[jax.ShapeDtypeStruct((E, d1), jnp.int32),
                  jax.ShapeDtypeStruct((E, d2), jnp.int32)],
        scratch_types=[pltpu.VMEM((CH,), jnp.int32),
                       pltpu.VMEM((CH,), jnp.int32),
                       pltpu.VMEM((CH, d1), jnp.int32),
                       pltpu.VMEM((CH, d2), jnp.int32),
                       pltpu.SemaphoreType.DMA],
    )
    def k(t1_hbm, t2_hbm, i1_hbm, i2_hbm, o1_hbm, o2_hbm,
          i1_v, i2_v, b1_v, b2_v, sem):
        cid = lax.axis_index("c")
        sid = lax.axis_index("s")
        wid = sid * NC + cid
        nw = NC * NS

        def body(i, carry):
            chunk = wid + i * nw

            @pl.when(chunk < NCHUNKS)
            def _():
                base = chunk * CH
                pltpu.sync_copy(i1_hbm.at[pl.ds(base, CH)], i1_v)
                pltpu.sync_copy(i2_hbm.at[pl.ds(base, CH)], i2_v)
                pltpu.async_copy(t1_hbm.at[i1_v], b1_v, sem).wait()
                pltpu.async_copy(t2_hbm.at[i2_v], b2_v, sem).wait()
                pltpu.sync_copy(b1_v, o1_hbm.at[pl.ds(base, CH)])
                pltpu.sync_copy(b2_v, o2_hbm.at[pl.ds(base, CH)])
            return carry

        lax.fori_loop(0, (NCHUNKS + nw - 1) // nw, body, 0)

    return k(t1, t2, idx1, idx2)


def _sc_scatter(msg3, dstv):
    mesh = plsc.VectorSubcoreMesh(core_axis_name="c", subcore_axis_name="s")
    rows_per_sub = NPAD // NS

    @functools.partial(
        pl.kernel, mesh=mesh,
        out_type=jax.ShapeDtypeStruct((NC, NPAD, HC), jnp.float32),
        scratch_types=[pltpu.VMEM((CH,), jnp.int32),
                       pltpu.VMEM((CH, HC), jnp.float32),
                       pltpu.VMEM_SHARED((NPAD, HC), jnp.float32),
                       pltpu.SemaphoreType.DMA],
    )
    def k(msg_hbm, dst_hbm, agg_hbm, idx_v, buf_v, acc_sh, sem):
        cid = lax.axis_index("c")
        sid = lax.axis_index("s")
        zer = jnp.zeros((16,), jnp.float32)

        def zrow(r, carry):
            for j in range(HC // 16):
                buf_v[r, j * 16:(j + 1) * 16] = zer
            return carry

        lax.fori_loop(0, CH, zrow, 0)

        def zcp(kk, carry):
            pltpu.sync_copy(buf_v, acc_sh.at[pl.ds(sid * rows_per_sub + kk * CH, CH)])
            return carry

        lax.fori_loop(0, rows_per_sub // CH, zcp, 0)
        plsc.subcore_barrier()

        def body(i, carry):
            chunk = sid + i * NS

            @pl.when(chunk < NCHUNKS)
            def _():
                base = chunk * CH
                pltpu.sync_copy(dst_hbm.at[pl.ds(base, CH)], idx_v)
                pltpu.sync_copy(msg_hbm.at[cid, pl.ds(base, CH)], buf_v)
                pltpu.sync_copy(buf_v, acc_sh.at[idx_v], add=True)
            return carry

        lax.fori_loop(0, (NCHUNKS + NS - 1) // NS, body, 0)
        plsc.subcore_barrier()

        def flsh(kk, carry):
            r0 = sid * rows_per_sub + kk * CH
            pltpu.sync_copy(acc_sh.at[pl.ds(r0, CH)], buf_v)
            pltpu.sync_copy(buf_v, agg_hbm.at[cid, pl.ds(r0, CH)])
            return carry

        lax.fori_loop(0, rows_per_sub // CH, flsh, 0)

    return k(msg3, dstv)


def _sc_scatter_mols(gated3, ids):
    mesh = plsc.VectorSubcoreMesh(core_axis_name="c", subcore_axis_name="s")

    @functools.partial(
        pl.kernel, mesh=mesh,
        out_type=jax.ShapeDtypeStruct((NC, NUM_MOLS, HC), jnp.float32),
        scratch_types=[pltpu.VMEM((CH,), jnp.int32),
                       pltpu.VMEM((CH, HC), jnp.float32),
                       pltpu.VMEM_SHARED((NUM_MOLS, HC), jnp.float32),
                       pltpu.SemaphoreType.DMA],
    )
    def k(g_hbm, ids_hbm, sums_hbm, idx_v, buf_v, acc_sh, sem):
        cid = lax.axis_index("c")
        sid = lax.axis_index("s")
        zer = jnp.zeros((16,), jnp.float32)

        def zrow(r, carry):
            for j in range(HC // 16):
                buf_v[r, j * 16:(j + 1) * 16] = zer
            return carry

        lax.fori_loop(0, CH, zrow, 0)

        @pl.when(sid < NUM_MOLS // CH)
        def _():
            pltpu.sync_copy(buf_v, acc_sh.at[pl.ds(sid * CH, CH)])

        plsc.subcore_barrier()

        def body(i, carry):
            chunk = sid + i * NS

            @pl.when(chunk < NCHUNKS)
            def _():
                base = chunk * CH
                pltpu.sync_copy(ids_hbm.at[pl.ds(base, CH)], idx_v)
                pltpu.sync_copy(g_hbm.at[cid, pl.ds(base, CH)], buf_v)
                pltpu.sync_copy(buf_v, acc_sh.at[idx_v], add=True)
            return carry

        lax.fori_loop(0, (NCHUNKS + NS - 1) // NS, body, 0)
        plsc.subcore_barrier()

        @pl.when(sid < NUM_MOLS // CH)
        def _():
            pltpu.sync_copy(acc_sh.at[pl.ds(sid * CH, CH)], buf_v)
            pltpu.sync_copy(buf_v, sums_hbm.at[cid, pl.ds(sid * CH, CH)])

    return k(gated3, ids)


def _b2(b):
    return jnp.tile(b[None, :], (8, 1))


def kernel(atom_features, bond_features, edge_index, bond_mol_ids, params):
    src = edge_index[0]
    dst = edge_index[1]
    x0 = jnp.pad(atom_features, ((0, NPAD - N), (0, 128 - atom_features.shape[1])))
    hb = jnp.pad(bond_features, ((0, 0), (0, 128 - bond_features.shape[1])))
    ts = None
    agg = None
    for li, lp in enumerate(params["layers"]):
        wcat = jnp.concatenate([lp["V"]["w"], lp["W_nei"]["w"],
                                lp["W"]["w"], lp["W_self"]["w"]], axis=1)
        bcat = jnp.concatenate([lp["V"]["b"], lp["W_nei"]["b"],
                                lp["W"]["b"], lp["W_self"]["b"]])
        if li == 0:
            wcat = jnp.pad(wcat, ((0, 128 - wcat.shape[0]), (0, 0)))
            tsn, tw, tself = _tables_call(True, False, x0, None, wcat, _b2(bcat))
        else:
            tsn, tw, tself = _tables_call(False, False, ts, agg, wcat, _b2(bcat))
        gsn, gw = _sc_gather(tsn, tw, src, dst, H, HC)
        wb = lp["W_bond"]["w"]
        if li == 0:
            wb = jnp.pad(wb, ((0, 128 - wb.shape[0]), (0, 0)))
        nb, msg3 = _edges_call(hb, wb, _b2(lp["W_bond"]["b"]), gsn, gw)
        agg = _sc_scatter(msg3, dst)
        ts = tself
        hb = nb
    wvw = jnp.concatenate([params["V"]["w"], params["W"]["w"]], axis=1)
    bvw = jnp.concatenate([params["V"]["b"], params["W"]["b"]])
    tv, tw2 = _tables_call(False, True, ts, agg, wvw, _b2(bvw))
    gv, gw2 = _sc_gather(tv, tw2, src, dst, HC, HC)
    wua = jnp.concatenate([params["U"]["w"], params["A"]["w"]], axis=1)
    bua = jnp.concatenate([params["U"]["b"], params["A"]["b"]])
    gated3, counts = _fedges_call(hb, wua, _b2(bua), gv, gw2,
                                  bond_mol_ids[:, None])
    sums3 = _sc_scatter_mols(gated3, bond_mol_ids)
    return _div_call(sums3, counts)
